# baseline (speedup 1.0000x reference)
import functools

import jax
import jax.numpy as jnp
from jax import lax
from jax.experimental import pallas as pl
from jax.experimental.pallas import tpu as pltpu
from jax.experimental.pallas import tpu_sc as plsc

NC, NS = 2, 16
NW = NC * NS
CHUNK = 128
NBUF = 2
SUB = 4


def _emb_body(table_hbm, idx_hbm, out_hbm, idx_v, rows_v, gsem, osem):
    nw, nchunk, chunk = idx_hbm.shape
    wid = lax.axis_index("s") * NC + lax.axis_index("c")
    base = wid * nchunk * chunk

    pltpu.sync_copy(idx_hbm.at[wid], idx_v)

    def gstart(i, b):
        pltpu.async_copy(
            table_hbm.at[idx_v.at[i]],
            rows_v.at[b], gsem.at[b])

    def gwait(b):
        pltpu.make_async_copy(
            table_hbm.at[idx_v.at[0]],
            rows_v.at[b], gsem.at[b]).wait()

    def ostart(i, b):
        pltpu.async_copy(
            rows_v.at[b],
            out_hbm.at[pl.ds(base + i * CHUNK, CHUNK)], osem.at[b])

    def owait(b):
        pltpu.make_async_copy(
            rows_v.at[b],
            out_hbm.at[pl.ds(base, CHUNK)], osem.at[b]).wait()

    for b in range(NBUF):
        gstart(b, b)

    @pl.loop(0, nchunk // NBUF)
    def _(g):
        i0 = g * NBUF
        for b in range(NBUF):
            i = i0 + b
            gwait(b)
            ostart(i, b)
            owait(b)
            nxt = i + NBUF

            @pl.when(nxt < nchunk)
            def _():
                gstart(nxt, b)


def _make_kernel(n_idx, vocab, d_pad):
    bpw = n_idx // NW
    nchunk = bpw // CHUNK
    mesh = plsc.VectorSubcoreMesh(
        core_axis_name="c", subcore_axis_name="s",
        num_cores=NC, num_subcores=NS)
    return pl.kernel(
        _emb_body,
        out_type=jax.ShapeDtypeStruct((n_idx, d_pad), jnp.float32),
        mesh=mesh,
        scratch_types=[
            pltpu.VMEM((nchunk, CHUNK), jnp.int32),
            pltpu.VMEM((NBUF, CHUNK, d_pad), jnp.float32),
            pltpu.SemaphoreType.DMA((NBUF,)),
            pltpu.SemaphoreType.DMA((NBUF,)),
        ],
        compiler_params=pltpu.CompilerParams(use_tc_tiling_on_sc=False),
    )


def kernel(input_ids, table):
    bsz, seq = input_ids.shape
    n_idx = bsz * seq
    vocab, d = table.shape
    d_pad = (d + 7) // 8 * 8
    if d_pad != d:
        table = jnp.pad(table, ((0, 0), (0, d_pad - d)))
    bpw = n_idx // NW
    ids_3d = input_ids.reshape(NW, bpw // CHUNK, CHUNK).astype(jnp.int32)
    out = _make_kernel(n_idx, vocab, d_pad)(table, ids_3d)
    return out[:, :d].reshape(bsz, seq, d)


# speedup vs baseline: 1.6068x; 1.6068x over previous
import functools

import jax
import jax.numpy as jnp
from jax import lax
from jax.experimental import pallas as pl
from jax.experimental.pallas import tpu as pltpu
from jax.experimental.pallas import tpu_sc as plsc

NC, NS = 2, 16
NW = NC * NS
CHUNK = 128
NBUF = 2


def _emb_body(table_hbm, idx_hbm, out_hbm, idx_v, rows_v, gsem, osem):
    nw, nchunk, chunk = idx_hbm.shape
    wid = lax.axis_index("s") * NC + lax.axis_index("c")
    base = wid * nchunk * chunk

    pltpu.sync_copy(idx_hbm.at[wid], idx_v)

    def gstart(i, b):
        pltpu.async_copy(
            table_hbm.at[idx_v.at[i]],
            rows_v.at[b], gsem.at[b])

    def gwait(b):
        pltpu.make_async_copy(
            table_hbm.at[idx_v.at[0]],
            rows_v.at[b], gsem.at[b]).wait()

    def ostart(i, b):
        pltpu.async_copy(
            rows_v.at[b],
            out_hbm.at[pl.ds(base + i * CHUNK, CHUNK)], osem.at[b])

    def owait(b):
        pltpu.make_async_copy(
            rows_v.at[b],
            out_hbm.at[pl.ds(base, CHUNK)], osem.at[b]).wait()

    for b in range(NBUF):
        gstart(b, b)

    @pl.loop(0, nchunk // NBUF)
    def _(g):
        i0 = g * NBUF
        for b in range(NBUF):
            i = i0 + b
            gwait(b)
            ostart(i, b)
            owait(b)
            nxt = i + NBUF

            @pl.when(nxt < nchunk)
            def _():
                gstart(nxt, b)


def _make_kernel(n_idx, vocab, d_pad):
    bpw = n_idx // NW
    nchunk = bpw // CHUNK
    mesh = plsc.VectorSubcoreMesh(
        core_axis_name="c", subcore_axis_name="s",
        num_cores=NC, num_subcores=NS)
    return pl.kernel(
        _emb_body,
        out_type=jax.ShapeDtypeStruct((n_idx, d_pad), jnp.float32),
        mesh=mesh,
        scratch_types=[
            pltpu.VMEM((nchunk, CHUNK), jnp.int32),
            pltpu.VMEM((NBUF, CHUNK, d_pad), jnp.float32),
            pltpu.SemaphoreType.DMA((NBUF,)),
            pltpu.SemaphoreType.DMA((NBUF,)),
        ],
        compiler_params=pltpu.CompilerParams(use_tc_tiling_on_sc=True),
    )


def kernel(input_ids, table):
    bsz, seq = input_ids.shape
    n_idx = bsz * seq
    vocab, d = table.shape
    d_pad = (d + 127) // 128 * 128
    if d_pad != d:
        table = jnp.pad(table, ((0, 0), (0, d_pad - d)))
    bpw = n_idx // NW
    ids_3d = input_ids.reshape(NW, bpw // CHUNK, CHUNK).astype(jnp.int32)
    out = _make_kernel(n_idx, vocab, d_pad)(table, ids_3d)
    return out[:, :d].reshape(bsz, seq, d)
